# baseline (speedup 1.0000x reference)
import functools
import math

import ml_dtypes
import numpy as np
import jax
import jax.numpy as jnp
from jax import lax
from jax.experimental import pallas as pl
from jax.experimental.pallas import tpu as pltpu
from jax.experimental.pallas import tpu_sc as plsc

DIM = 1024
SEQ = 2048
BATCH = 4
SCALE = math.sqrt(DIM)

NC, NS, L = 2, 16, 16
NW = NC * NS
PPW = SEQ // NW
CHP = 8
CHR = CHP * BATCH
NCHUNK = PPW // CHP
VPR = DIM // L
CPR = DIM // (2 * L)
NBUF = 2


def _pos_enc_bf16() -> np.ndarray:
    pos = np.arange(SEQ, dtype=np.float64)[:, None]
    idx = np.arange(0, DIM, 2, dtype=np.float64)[None, :]
    angle = pos / (10000.0 ** (idx / DIM))
    pe = np.zeros((SEQ, DIM), dtype=np.float32)
    pe[:, 0::2] = np.sin(angle)
    pe[:, 1::2] = np.cos(angle)
    bits = pe.astype(ml_dtypes.bfloat16).view(np.uint16).astype(np.uint32)
    b4 = bits.reshape(SEQ, CPR, 2, L)
    words = (b4[:, :, 1, :] << 16) | b4[:, :, 0, :]
    return words.reshape(-1).view(np.int32)


_PE = _pos_enc_bf16()


PED = CHP * DIM // 2


def _emb_body(x_hbm, tab_hbm, pe_hbm, out_hbm,
              idx_v, buf, pe_v, gsem, psem, osem):
    wid = lax.axis_index("s") * NC + lax.axis_index("c")
    p0 = wid * PPW

    for b in range(BATCH):
        pltpu.sync_copy(x_hbm.at[b, pl.ds(p0, PPW)], idx_v.at[b])

    def start_chunk(j, slot):
        for b in range(BATCH):
            pltpu.async_copy(
                tab_hbm.at[idx_v.at[b, pl.ds(j * CHP, CHP)]],
                buf.at[slot, pl.ds(b * CHP, CHP)], gsem)
        pltpu.async_copy(
            pe_hbm.at[pl.ds((p0 + j * CHP) * (DIM // 2), PED)],
            pe_v.at[pl.ds(slot * PED, PED)], psem)

    def wait_in(slot):
        pltpu.make_async_copy(tab_hbm.at[pl.ds(0, CHR)], buf.at[slot],
                              gsem).wait()
        pltpu.make_async_copy(pe_hbm.at[pl.ds(0, PED)],
                              pe_v.at[pl.ds(slot * PED, PED)], psem).wait()

    def store_chunk(j, slot):
        for b in range(BATCH):
            pltpu.async_copy(
                buf.at[slot, pl.ds(b * CHP, CHP)],
                out_hbm.at[pl.ds(b * SEQ + p0 + j * CHP, CHP)],
                osem)

    def wait_out(slot):
        pltpu.make_async_copy(buf.at[slot], out_hbm.at[pl.ds(0, CHR)],
                              osem).wait()

    def compute_chunk(j, slot):
        @plsc.parallel_loop(0, CHP * CPR, unroll=2)
        def _body(i):
            p = lax.shift_right_logical(i, 5)
            c2 = lax.bitwise_and(i, CPR - 1)
            w = pe_v[pl.ds(slot * PED + i * L, L)]
            lo = lax.bitcast_convert_type(lax.shift_left(w, 16),
                                          jnp.float32)
            hi = lax.bitcast_convert_type(
                lax.bitwise_and(w, jnp.int32(-65536)), jnp.float32)
            sl_lo = pl.ds(c2 * 2 * L, L)
            sl_hi = pl.ds(c2 * 2 * L + L, L)
            for b in range(BATCH):
                r = b * CHP + p
                buf[slot, r, sl_lo] = buf[slot, r, sl_lo] * SCALE + lo
                buf[slot, r, sl_hi] = buf[slot, r, sl_hi] * SCALE + hi

    start_chunk(0, 0)

    def _pipeline(j, _):
        slot = lax.bitwise_and(j, 1)
        nxt = 1 - slot
        wait_in(slot)

        @pl.when(j < NCHUNK - 1)
        def _prefetch():
            @pl.when(j >= 1)
            def _drain():
                wait_out(nxt)
            start_chunk(j + 1, nxt)

        compute_chunk(j, slot)
        store_chunk(j, slot)
        return 0
    lax.fori_loop(0, NCHUNK, _pipeline, 0)
    wait_out(0)
    wait_out(1)


@jax.jit
def kernel(x, emb_table):
    mesh = plsc.VectorSubcoreMesh(core_axis_name="c", subcore_axis_name="s")
    run = functools.partial(
        pl.kernel,
        out_type=jax.ShapeDtypeStruct((BATCH * SEQ, DIM), jnp.float32),
        mesh=mesh,
        scratch_types=[
            pltpu.VMEM((BATCH, PPW), jnp.int32),
            pltpu.VMEM((NBUF, CHR, DIM), jnp.float32),
            pltpu.VMEM((NBUF * CHP * DIM // 2,), jnp.int32),
        ] + [pltpu.SemaphoreType.DMA] * 3,
    )(_emb_body)
    out = run(x, emb_table, _PE)
    return out.reshape(BATCH, SEQ, DIM)


# speedup vs baseline: 1.2879x; 1.2879x over previous
import functools
import math

import ml_dtypes
import numpy as np
import jax
import jax.numpy as jnp
from jax import lax
from jax.experimental import pallas as pl
from jax.experimental.pallas import tpu as pltpu
from jax.experimental.pallas import tpu_sc as plsc

DIM = 1024
SEQ = 2048
BATCH = 4
SCALE = math.sqrt(DIM)

NC, NS, L = 2, 16, 16
NW = NC * NS
PPW = SEQ // NW
CHP = 8
CHR = CHP * BATCH
NCHUNK = PPW // CHP
VPR = DIM // L
CPR = DIM // (2 * L)
NBUF = 2


def _pos_enc_bf16() -> np.ndarray:
    pos = np.arange(SEQ, dtype=np.float64)[:, None]
    idx = np.arange(0, DIM, 2, dtype=np.float64)[None, :]
    angle = pos / (10000.0 ** (idx / DIM))
    pe = np.zeros((SEQ, DIM), dtype=np.float32)
    pe[:, 0::2] = np.sin(angle)
    pe[:, 1::2] = np.cos(angle)
    bits = pe.astype(ml_dtypes.bfloat16).view(np.uint16).astype(np.uint32)
    b4 = bits.reshape(SEQ, CPR, 2, L)
    words = (b4[:, :, 1, :] << 16) | b4[:, :, 0, :]
    return words.reshape(-1).view(np.int32)


_PE = _pos_enc_bf16()


PED = CHP * DIM // 2


def _emb_body(x_hbm, tab_hbm, pe_hbm, out_hbm,
              idx_v, buf, pe_v, gsem, psem, osem):
    wid = lax.axis_index("s") * NC + lax.axis_index("c")
    p0 = wid * PPW

    for b in range(BATCH):
        pltpu.sync_copy(x_hbm.at[b, pl.ds(p0, PPW)], idx_v.at[b])

    def start_chunk(j, slot):
        for b in range(BATCH):
            pltpu.async_copy(
                tab_hbm.at[idx_v.at[b, pl.ds(j * CHP, CHP)]],
                buf.at[slot, pl.ds(b * CHP, CHP)], gsem)
        pltpu.async_copy(
            pe_hbm.at[pl.ds((p0 + j * CHP) * (DIM // 2), PED)],
            pe_v.at[pl.ds(slot * PED, PED)], psem)

    def wait_in(slot):
        pltpu.make_async_copy(tab_hbm.at[pl.ds(0, CHR)], buf.at[slot],
                              gsem).wait()
        pltpu.make_async_copy(pe_hbm.at[pl.ds(0, PED)],
                              pe_v.at[pl.ds(slot * PED, PED)], psem).wait()

    def store_chunk(j, slot):
        for b in range(BATCH):
            pltpu.async_copy(
                buf.at[slot, pl.ds(b * CHP, CHP)],
                out_hbm.at[pl.ds(b * SEQ + p0 + j * CHP, CHP)],
                osem)

    def wait_out(slot):
        pltpu.make_async_copy(buf.at[slot], out_hbm.at[pl.ds(0, CHR)],
                              osem).wait()

    def compute_chunk(j, slot):
        @plsc.parallel_loop(0, CHP * CPR, unroll=8)
        def _body(i):
            p = lax.shift_right_logical(i, 5)
            c2 = lax.bitwise_and(i, CPR - 1)
            w = pe_v[pl.ds(slot * PED + i * L, L)]
            lo = lax.bitcast_convert_type(lax.shift_left(w, 16),
                                          jnp.float32)
            hi = lax.bitcast_convert_type(
                lax.bitwise_and(w, jnp.int32(-65536)), jnp.float32)
            sl_lo = pl.ds(c2 * 2 * L, L)
            sl_hi = pl.ds(c2 * 2 * L + L, L)
            for b in range(BATCH):
                r = b * CHP + p
                buf[slot, r, sl_lo] = buf[slot, r, sl_lo] * SCALE + lo
                buf[slot, r, sl_hi] = buf[slot, r, sl_hi] * SCALE + hi

    start_chunk(0, 0)

    def _pipeline(j, _):
        slot = lax.bitwise_and(j, 1)
        nxt = 1 - slot
        wait_in(slot)

        @pl.when(j < NCHUNK - 1)
        def _prefetch():
            @pl.when(j >= 1)
            def _drain():
                wait_out(nxt)
            start_chunk(j + 1, nxt)

        compute_chunk(j, slot)
        store_chunk(j, slot)
        return 0
    lax.fori_loop(0, NCHUNK, _pipeline, 0)
    wait_out(0)
    wait_out(1)


@jax.jit
def kernel(x, emb_table):
    mesh = plsc.VectorSubcoreMesh(core_axis_name="c", subcore_axis_name="s")
    run = functools.partial(
        pl.kernel,
        out_type=jax.ShapeDtypeStruct((BATCH * SEQ, DIM), jnp.float32),
        mesh=mesh,
        scratch_types=[
            pltpu.VMEM((BATCH, PPW), jnp.int32),
            pltpu.VMEM((NBUF, CHR, DIM), jnp.float32),
            pltpu.VMEM((NBUF * CHP * DIM // 2,), jnp.int32),
        ] + [pltpu.SemaphoreType.DMA] * 3,
    )(_emb_body)
    out = run(x, emb_table, _PE)
    return out.reshape(BATCH, SEQ, DIM)


# speedup vs baseline: 1.3018x; 1.0108x over previous
import functools
import math

import ml_dtypes
import numpy as np
import jax
import jax.numpy as jnp
from jax import lax
from jax.experimental import pallas as pl
from jax.experimental.pallas import tpu as pltpu
from jax.experimental.pallas import tpu_sc as plsc

DIM = 1024
SEQ = 2048
BATCH = 4
SCALE = math.sqrt(DIM)

NC, NS, L = 2, 16, 16
NW = NC * NS
PPW = SEQ // NW
CHP = 8
CHR = CHP * BATCH
NCHUNK = PPW // CHP
VPR = DIM // L
CPR = DIM // (2 * L)
NBUF = 2


def _pos_enc_bf16() -> np.ndarray:
    pos = np.arange(SEQ, dtype=np.float64)[:, None]
    idx = np.arange(0, DIM, 2, dtype=np.float64)[None, :]
    angle = pos / (10000.0 ** (idx / DIM))
    pe = np.zeros((SEQ, DIM), dtype=np.float32)
    pe[:, 0::2] = np.sin(angle)
    pe[:, 1::2] = np.cos(angle)
    bits = pe.astype(ml_dtypes.bfloat16).view(np.uint16).astype(np.uint32)
    b4 = bits.reshape(SEQ, CPR, 2, L)
    words = (b4[:, :, 1, :] << 16) | b4[:, :, 0, :]
    return words.reshape(-1).view(np.int32)


_PE = _pos_enc_bf16()


PED = CHP * DIM // 2


def _emb_body(x_hbm, tab_hbm, pe_hbm, out_hbm,
              idx_v, buf, pe_v, gsem, psem, osem):
    wid = lax.axis_index("s") * NC + lax.axis_index("c")
    p0 = wid * PPW

    for b in range(BATCH):
        pltpu.sync_copy(x_hbm.at[b, pl.ds(p0, PPW)], idx_v.at[b])

    def start_chunk(j, slot):
        for b in range(BATCH):
            pltpu.async_copy(
                tab_hbm.at[idx_v.at[b, pl.ds(j * CHP, CHP)]],
                buf.at[slot, pl.ds(b * CHP, CHP)], gsem)
        pltpu.async_copy(
            pe_hbm.at[pl.ds((p0 + j * CHP) * (DIM // 2), PED)],
            pe_v.at[pl.ds(slot * PED, PED)], psem)

    def wait_in(slot):
        pltpu.make_async_copy(tab_hbm.at[pl.ds(0, CHR)], buf.at[slot],
                              gsem).wait()
        pltpu.make_async_copy(pe_hbm.at[pl.ds(0, PED)],
                              pe_v.at[pl.ds(slot * PED, PED)], psem).wait()

    def store_chunk(j, slot):
        for b in range(BATCH):
            pltpu.async_copy(
                buf.at[slot, pl.ds(b * CHP, CHP)],
                out_hbm.at[pl.ds(b * SEQ + p0 + j * CHP, CHP)],
                osem)

    def wait_out(slot):
        pltpu.make_async_copy(buf.at[slot], out_hbm.at[pl.ds(0, CHR)],
                              osem).wait()

    def compute_chunk(j, slot):
        @plsc.parallel_loop(0, CHP * CPR, unroll=4)
        def _body(i):
            p = lax.shift_right_logical(i, 5)
            c2 = lax.bitwise_and(i, CPR - 1)
            w = pe_v[pl.ds(slot * PED + i * L, L)]
            lo = lax.bitcast_convert_type(lax.shift_left(w, 16),
                                          jnp.float32)
            hi = lax.bitcast_convert_type(
                lax.bitwise_and(w, jnp.int32(-65536)), jnp.float32)
            sl_lo = pl.ds(c2 * 2 * L, L)
            sl_hi = pl.ds(c2 * 2 * L + L, L)
            for b in range(BATCH):
                r = b * CHP + p
                buf[slot, r, sl_lo] = buf[slot, r, sl_lo] * SCALE + lo
                buf[slot, r, sl_hi] = buf[slot, r, sl_hi] * SCALE + hi

    start_chunk(0, 0)

    def _pipeline(j, _):
        slot = lax.bitwise_and(j, 1)
        nxt = 1 - slot
        wait_in(slot)

        @pl.when(j < NCHUNK - 1)
        def _prefetch():
            @pl.when(j >= 1)
            def _drain():
                wait_out(nxt)
            start_chunk(j + 1, nxt)

        compute_chunk(j, slot)
        store_chunk(j, slot)
        return 0
    lax.fori_loop(0, NCHUNK, _pipeline, 0)
    wait_out(0)
    wait_out(1)


@jax.jit
def kernel(x, emb_table):
    mesh = plsc.VectorSubcoreMesh(core_axis_name="c", subcore_axis_name="s")
    run = functools.partial(
        pl.kernel,
        out_type=jax.ShapeDtypeStruct((BATCH * SEQ, DIM), jnp.float32),
        mesh=mesh,
        scratch_types=[
            pltpu.VMEM((BATCH, PPW), jnp.int32),
            pltpu.VMEM((NBUF, CHR, DIM), jnp.float32),
            pltpu.VMEM((NBUF * CHP * DIM // 2,), jnp.int32),
        ] + [pltpu.SemaphoreType.DMA] * 3,
    )(_emb_body)
    out = run(x, emb_table, _PE)
    return out.reshape(BATCH, SEQ, DIM)
